# baseline (speedup 1.0000x reference)
import functools
import math

import jax
import jax.numpy as jnp
from jax import lax
from jax.experimental import pallas as pl
from jax.experimental.pallas import tpu as pltpu
from jax.experimental.pallas import tpu_sc as plsc

B = 1024
N = 100000
D = 128
A = 1
TN = 2048
GRID = math.ceil(N / TN)
NEG = -10000000.0


def _gather_answer_rows(entity_embeddings, answers_flat):
    info = plsc.get_sparse_core_info()
    nc, ns = info.num_cores, info.num_subcores
    nw = nc * ns
    b_per_w = B // nw
    mesh = plsc.VectorSubcoreMesh(core_axis_name="c", subcore_axis_name="s")

    @functools.partial(
        pl.kernel,
        mesh=mesh,
        out_type=jax.ShapeDtypeStruct((B, D), jnp.float32),
        scratch_types=[
            pltpu.VMEM((b_per_w,), jnp.int32),
            pltpu.VMEM((b_per_w, D), jnp.float32),
            pltpu.SemaphoreType.DMA,
        ],
    )
    def gather_k(table_hbm, idx_hbm, out_hbm, idx_v, rows_v, sem):
        wid = lax.axis_index("s") * nc + lax.axis_index("c")
        base = wid * b_per_w
        pltpu.sync_copy(idx_hbm.at[pl.ds(base, b_per_w)], idx_v)
        pltpu.async_copy(table_hbm.at[idx_v], rows_v, sem).wait()
        pltpu.sync_copy(rows_v, out_hbm.at[pl.ds(base, b_per_w)])

    return gather_k(entity_embeddings, answers_flat)


def _rank_body(ans_ref, u_ref, q_ref, eans_ref, e_ref,
               out_ref, mrr_ref, h1_ref, h3_ref, h10_ref,
               s_scr, gt_scr, eq_scr):
    j = pl.program_id(0)

    @pl.when(j == 0)
    def _init():
        s_scr[...] = jnp.sum(q_ref[...] * eans_ref[...], axis=1, keepdims=True)
        gt_scr[...] = jnp.zeros((B, 1), jnp.float32)
        eq_scr[...] = jnp.zeros((B, 1), jnp.float32)

    q = q_ref[...]
    e = e_ref[...]
    qh = q.astype(jnp.bfloat16)
    ql = (q - qh.astype(jnp.float32)).astype(jnp.bfloat16)
    eh = e.astype(jnp.bfloat16)
    el = (e - eh.astype(jnp.float32)).astype(jnp.bfloat16)
    dims = (((1,), (1,)), ((), ()))
    scores = (lax.dot_general(qh, eh, dims, preferred_element_type=jnp.float32)
              + (lax.dot_general(qh, el, dims, preferred_element_type=jnp.float32)
                 + lax.dot_general(ql, eh, dims, preferred_element_type=jnp.float32)))
    out_ref[...] = scores

    one, zero = 1.0, 0.0

    @pl.when(j == GRID - 1)
    def _fin():
        add = (u_ref[...] * eq_scr[...]).astype(jnp.int32).astype(jnp.float32)
        rank = gt_scr[...] + 1.0 + add
        mrr_ref[...] = 1.0 / rank
        h1_ref[...] = jnp.where(rank < 1.5, one, zero)
        h3_ref[...] = jnp.where(rank < 3.5, one, zero)
        h10_ref[...] = jnp.where(rank < 10.5, one, zero)


def _score_and_rank(ans2d, u, q, eans, e):
    col_spec = pl.BlockSpec((B, 1), lambda j: (0, 0))
    return pl.pallas_call(
        _rank_body,
        grid=(GRID,),
        in_specs=[
            col_spec,
            col_spec,
            pl.BlockSpec((B, D), lambda j: (0, 0)),
            pl.BlockSpec((B, D), lambda j: (0, 0)),
            pl.BlockSpec((TN, D), lambda j: (j, 0)),
        ],
        out_specs=[
            pl.BlockSpec((B, TN), lambda j: (0, j)),
            col_spec, col_spec, col_spec, col_spec,
        ],
        out_shape=[
            jax.ShapeDtypeStruct((B, N), jnp.float32),
            jax.ShapeDtypeStruct((B, 1), jnp.float32),
            jax.ShapeDtypeStruct((B, 1), jnp.float32),
            jax.ShapeDtypeStruct((B, 1), jnp.float32),
            jax.ShapeDtypeStruct((B, 1), jnp.float32),
        ],
        scratch_shapes=[
            pltpu.VMEM((B, 1), jnp.float32),
            pltpu.VMEM((B, 1), jnp.float32),
            pltpu.VMEM((B, 1), jnp.float32),
        ],
        compiler_params=pltpu.CompilerParams(
            dimension_semantics=("arbitrary",)),
    )(ans2d, u, q, eans, e)


def kernel(query_encoding, entity_embeddings, entailed_answers):
    eans = _gather_answer_rows(entity_embeddings, entailed_answers.reshape(B))
    u = jax.random.uniform(jax.random.key(42), (B, A), dtype=jnp.float32)
    all_scoring, mrr, h1, h3, h10 = _score_and_rank(
        entailed_answers, u, query_encoding, eans, entity_embeddings)
    return (all_scoring, mrr.reshape(B), h1.reshape(B), h3.reshape(B),
            h10.reshape(B))


# speedup vs baseline: 1.0237x; 1.0237x over previous
import functools
import math

import jax
import jax.numpy as jnp
from jax import lax
from jax.experimental import pallas as pl
from jax.experimental.pallas import tpu as pltpu
from jax.experimental.pallas import tpu_sc as plsc

B = 1024
N = 100000
D = 128
A = 1
TN = 2048
GRID = math.ceil(N / TN)
NEG = -10000000.0


def _gather_answer_rows(entity_embeddings, answers_flat):
    info = plsc.get_sparse_core_info()
    nc, ns = info.num_cores, info.num_subcores
    nw = nc * ns
    b_per_w = B // nw
    mesh = plsc.VectorSubcoreMesh(core_axis_name="c", subcore_axis_name="s")

    @functools.partial(
        pl.kernel,
        mesh=mesh,
        out_type=jax.ShapeDtypeStruct((B, D), jnp.float32),
        scratch_types=[
            pltpu.VMEM((b_per_w,), jnp.int32),
            pltpu.VMEM((b_per_w, D), jnp.float32),
            pltpu.SemaphoreType.DMA,
        ],
    )
    def gather_k(table_hbm, idx_hbm, out_hbm, idx_v, rows_v, sem):
        wid = lax.axis_index("s") * nc + lax.axis_index("c")
        base = wid * b_per_w
        pltpu.sync_copy(idx_hbm.at[pl.ds(base, b_per_w)], idx_v)
        pltpu.async_copy(table_hbm.at[idx_v], rows_v, sem).wait()
        pltpu.sync_copy(rows_v, out_hbm.at[pl.ds(base, b_per_w)])

    return gather_k(entity_embeddings, answers_flat)


def _answer_score_body(q_ref, eans_ref, s_ref):
    qe = lax.dot_general(q_ref[...], eans_ref[...], (((1,), (1,)), ((), ())),
                         preferred_element_type=jnp.float32)
    r_i = lax.broadcasted_iota(jnp.int32, (B, B), 0)
    c_i = lax.broadcasted_iota(jnp.int32, (B, B), 1)
    s_ref[...] = jnp.sum(jnp.where(r_i == c_i, qe, 0.0),
                         axis=1, keepdims=True)


def _answer_scores(q, eans):
    return pl.pallas_call(
        _answer_score_body,
        out_shape=jax.ShapeDtypeStruct((B, 1), jnp.float32),
    )(q, eans)


def _rank_body(ans_ref, u_ref, s_ref, q_ref, e_ref,
               out_ref, mrr_ref, h1_ref, h3_ref, h10_ref,
               gt_scr, eq_scr):
    j = pl.program_id(0)
    dims = (((1,), (1,)), ((), ()))
    one, zero = 1.0, 0.0

    @pl.when(j == 0)
    def _init():
        gt_scr[...] = jnp.zeros((B, 1), jnp.float32)
        eq_scr[...] = jnp.zeros((B, 1), jnp.float32)

    scores = lax.dot_general(
        q_ref[...], e_ref[...], dims,
        preferred_element_type=jnp.float32)
    out_ref[...] = scores
    s = s_ref[...]

    @pl.when(j < GRID - 1)
    def _count():
        gt_scr[...] += jnp.sum(jnp.where(scores > s, one, zero),
                               axis=1, keepdims=True)
        eq_scr[...] += jnp.sum(jnp.where(scores == s, one, zero),
                               axis=1, keepdims=True)

    @pl.when(j == GRID - 1)
    def _fin():
        col = j * TN + lax.broadcasted_iota(jnp.int32, (B, TN), 1)
        sm = jnp.where(col < N, scores, -jnp.inf)
        gt = gt_scr[...] + jnp.sum(jnp.where(sm > s, one, zero),
                                   axis=1, keepdims=True)
        eq = eq_scr[...] + jnp.sum(jnp.where(sm == s, one, zero),
                                   axis=1, keepdims=True)
        eq = eq - one + jnp.where(s == NEG, one, zero)
        gt = gt + jnp.where(s < NEG, one, zero)
        add = (u_ref[...] * eq).astype(jnp.int32).astype(jnp.float32)
        rank = gt + 1.0 + add
        mrr_ref[...] = 1.0 / rank
        h1_ref[...] = jnp.where(rank < 1.5, one, zero)
        h3_ref[...] = jnp.where(rank < 3.5, one, zero)
        h10_ref[...] = jnp.where(rank < 10.5, one, zero)


def _score_and_rank(ans2d, u, s, q, e):
    col_spec = pl.BlockSpec((B, 1), lambda j: (0, 0))
    return pl.pallas_call(
        _rank_body,
        grid=(GRID,),
        in_specs=[
            col_spec,
            col_spec,
            col_spec,
            pl.BlockSpec((B, D), lambda j: (0, 0)),
            pl.BlockSpec((TN, D), lambda j: (j, 0)),
        ],
        out_specs=[
            pl.BlockSpec((B, TN), lambda j: (0, j)),
            col_spec, col_spec, col_spec, col_spec,
        ],
        out_shape=[
            jax.ShapeDtypeStruct((B, N), jnp.float32),
            jax.ShapeDtypeStruct((B, 1), jnp.float32),
            jax.ShapeDtypeStruct((B, 1), jnp.float32),
            jax.ShapeDtypeStruct((B, 1), jnp.float32),
            jax.ShapeDtypeStruct((B, 1), jnp.float32),
        ],
        scratch_shapes=[
            pltpu.VMEM((B, 1), jnp.float32),
            pltpu.VMEM((B, 1), jnp.float32),
        ],
        compiler_params=pltpu.CompilerParams(
            dimension_semantics=("arbitrary",)),
    )(ans2d, u, s, q, e)


def kernel(query_encoding, entity_embeddings, entailed_answers):
    eans = _gather_answer_rows(entity_embeddings, entailed_answers.reshape(B))
    s = _answer_scores(query_encoding, eans)
    u = jax.random.uniform(jax.random.key(42), (B, A), dtype=jnp.float32)
    all_scoring, mrr, h1, h3, h10 = _score_and_rank(
        entailed_answers, u, s, query_encoding, entity_embeddings)
    return (all_scoring, mrr.reshape(B), h1.reshape(B), h3.reshape(B),
            h10.reshape(B))
